# baseline (speedup 1.0000x reference)
import functools

import jax
import jax.numpy as jnp
from jax import lax
from jax.experimental import pallas as pl
from jax.experimental.pallas import tpu as pltpu
from jax.experimental.pallas import tpu_sc as plsc

F = 26
B = 16384
V = 100000
D = 16

NC = 2
NS = 16
NW = NC * NS

ROWS = F * B
RPW = ROWS // NW
CHUNK = 128
NCH = RPW // CHUNK
K = 4
NG = NCH // K
NPAIR = NG // 2


def _gather_body(table_hbm, idx_hbm, out_hbm, idx_v, *scratch):
    bufs = scratch[: 2 * K]
    gsem0, gsem1, wsem0, wsem1 = scratch[2 * K :]
    gsems = (gsem0, gsem1)
    wsems = (wsem0, wsem1)

    wid = lax.axis_index("s") * NC + lax.axis_index("c")
    base = wid * RPW

    pltpu.sync_copy(idx_hbm.at[wid], idx_v)

    def start_group(g, p):
        for k in range(K):
            j = g * K + k
            pltpu.make_async_copy(
                table_hbm.at[idx_v.at[j]], bufs[p * K + k], gsems[p]
            ).start()

    def drain_gathers(g, p):
        for k in range(K):
            j = g * K + k
            pltpu.make_async_copy(
                table_hbm.at[idx_v.at[j]], bufs[p * K + k], gsems[p]
            ).wait()

    def start_writes(g, p):
        for k in range(K):
            j = g * K + k
            pltpu.make_async_copy(
                bufs[p * K + k],
                out_hbm.at[pl.ds(base + j * CHUNK, CHUNK)],
                wsems[p],
            ).start()

    def drain_writes(g, p):
        for k in range(K):
            j = g * K + k
            pltpu.make_async_copy(
                bufs[p * K + k],
                out_hbm.at[pl.ds(base + j * CHUNK, CHUNK)],
                wsems[p],
            ).wait()

    start_group(0, 0)
    start_group(1, 1)

    def pair_body(h, carry):
        g0 = 2 * h
        drain_gathers(g0, 0)
        start_writes(g0, 0)
        drain_writes(g0, 0)
        start_group(g0 + 2, 0)
        drain_gathers(g0 + 1, 1)
        start_writes(g0 + 1, 1)
        drain_writes(g0 + 1, 1)
        start_group(g0 + 3, 1)
        return carry

    lax.fori_loop(0, NPAIR - 1, pair_body, 0, unroll=False)

    gl = NG - 2
    drain_gathers(gl, 0)
    start_writes(gl, 0)
    drain_writes(gl, 0)
    drain_gathers(gl + 1, 1)
    start_writes(gl + 1, 1)
    drain_writes(gl + 1, 1)


def kernel(tables, indices):
    flat_tables = tables.reshape(F * V, D)
    offs = (jnp.arange(F, dtype=jnp.int32) * V)[:, None]
    flat_idx = (indices + offs).T.reshape(NW, NCH, CHUNK)

    mesh = plsc.VectorSubcoreMesh(core_axis_name="c", subcore_axis_name="s")
    run = functools.partial(
        pl.kernel,
        mesh=mesh,
        compiler_params=pltpu.CompilerParams(use_tc_tiling_on_sc=False),
        out_type=jax.ShapeDtypeStruct((ROWS, D), jnp.float32),
        scratch_types=[
            pltpu.VMEM((NCH, CHUNK), jnp.int32),
            *[pltpu.VMEM((CHUNK, D), jnp.float32) for _ in range(2 * K)],
            pltpu.SemaphoreType.DMA,
            pltpu.SemaphoreType.DMA,
            pltpu.SemaphoreType.DMA,
            pltpu.SemaphoreType.DMA,
        ],
    )(_gather_body)
    out = run(flat_tables, flat_idx)
    return out.reshape(B, F, D)


# speedup vs baseline: 1.0003x; 1.0003x over previous
import functools

import jax
import jax.numpy as jnp
from jax import lax
from jax.experimental import pallas as pl
from jax.experimental.pallas import tpu as pltpu
from jax.experimental.pallas import tpu_sc as plsc

F = 26
B = 16384
V = 100000
D = 16

NC = 2
NS = 16
NW = NC * NS

ROWS = F * B
RPW = ROWS // NW
CHUNK = 512
NCH = RPW // CHUNK
K = 1
NG = NCH // K
NPAIR = NG // 2


def _gather_body(table_hbm, idx_hbm, out_hbm, idx_v, *scratch):
    bufs = scratch[: 2 * K]
    gsem0, gsem1, wsem0, wsem1 = scratch[2 * K :]
    gsems = (gsem0, gsem1)
    wsems = (wsem0, wsem1)

    wid = lax.axis_index("s") * NC + lax.axis_index("c")
    base = wid * RPW

    pltpu.sync_copy(idx_hbm.at[wid], idx_v)

    def start_group(g, p):
        for k in range(K):
            j = g * K + k
            pltpu.make_async_copy(
                table_hbm.at[idx_v.at[j]], bufs[p * K + k], gsems[p]
            ).start()

    def drain_gathers(g, p):
        for k in range(K):
            j = g * K + k
            pltpu.make_async_copy(
                table_hbm.at[idx_v.at[j]], bufs[p * K + k], gsems[p]
            ).wait()

    def start_writes(g, p):
        for k in range(K):
            j = g * K + k
            pltpu.make_async_copy(
                bufs[p * K + k],
                out_hbm.at[pl.ds(base + j * CHUNK, CHUNK)],
                wsems[p],
            ).start()

    def drain_writes(g, p):
        for k in range(K):
            j = g * K + k
            pltpu.make_async_copy(
                bufs[p * K + k],
                out_hbm.at[pl.ds(base + j * CHUNK, CHUNK)],
                wsems[p],
            ).wait()

    start_group(0, 0)
    start_group(1, 1)

    def pair_body(h, carry):
        g0 = 2 * h
        drain_gathers(g0, 0)
        start_writes(g0, 0)
        drain_writes(g0, 0)
        start_group(g0 + 2, 0)
        drain_gathers(g0 + 1, 1)
        start_writes(g0 + 1, 1)
        drain_writes(g0 + 1, 1)
        start_group(g0 + 3, 1)
        return carry

    lax.fori_loop(0, NPAIR - 1, pair_body, 0, unroll=False)

    gl = NG - 2
    drain_gathers(gl, 0)
    start_writes(gl, 0)
    drain_writes(gl, 0)
    drain_gathers(gl + 1, 1)
    start_writes(gl + 1, 1)
    drain_writes(gl + 1, 1)


def kernel(tables, indices):
    flat_tables = tables.reshape(F * V, D)
    offs = (jnp.arange(F, dtype=jnp.int32) * V)[:, None]
    flat_idx = (indices + offs).T.reshape(NW, NCH, CHUNK)

    mesh = plsc.VectorSubcoreMesh(core_axis_name="c", subcore_axis_name="s")
    run = functools.partial(
        pl.kernel,
        mesh=mesh,
        compiler_params=pltpu.CompilerParams(use_tc_tiling_on_sc=False),
        out_type=jax.ShapeDtypeStruct((ROWS, D), jnp.float32),
        scratch_types=[
            pltpu.VMEM((NCH, CHUNK), jnp.int32),
            *[pltpu.VMEM((CHUNK, D), jnp.float32) for _ in range(2 * K)],
            pltpu.SemaphoreType.DMA,
            pltpu.SemaphoreType.DMA,
            pltpu.SemaphoreType.DMA,
            pltpu.SemaphoreType.DMA,
        ],
    )(_gather_body)
    out = run(flat_tables, flat_idx)
    return out.reshape(B, F, D)
